# baseline (speedup 1.0000x reference)
import functools

import jax
import jax.numpy as jnp
from jax import lax
from jax.experimental import pallas as pl
from jax.experimental.pallas import tpu as pltpu
from jax.experimental.pallas import tpu_sc as plsc

N = 10000
NP = 10240
E = 320000
EP = 327680
D = 128
NC = 2
NS = 16
NW = NC * NS
C = 128
CH = EP // (NW * C)
RPT = NP // NS

_mesh = plsc.VectorSubcoreMesh(core_axis_name="c", subcore_axis_name="s")


CH2 = EP // (NS * C)


def _deg_kernel_body(idx_hbm, ones_hbm, zrow_hbm, out_hbm,
                     cidx, ones_v, hist):
    c = lax.axis_index("c")
    s = lax.axis_index("s")
    pltpu.sync_copy(zrow_hbm, hist.at[pl.ds(s * RPT, RPT)])
    pltpu.sync_copy(ones_hbm, ones_v)
    pltpu.sync_copy(idx_hbm.at[c, s], cidx)
    plsc.subcore_barrier()

    def body(j, carry):
        pltpu.sync_copy(ones_v, hist.at[cidx.at[j]], add=True)
        return carry

    lax.fori_loop(0, CH2, body, 0)
    plsc.subcore_barrier()
    pltpu.sync_copy(hist.at[pl.ds(s * RPT, RPT)],
                    out_hbm.at[c, pl.ds(s * RPT, RPT)])


CH_A = 160
CH_B = 0
M2 = 16 * CH_A + 16 * CH_B + (CH_A - CH_B)


CB = 32


def _agg_kernel_body(src_hbm, dst_hbm, tab_hbm, zrow_hbm, out_hbm,
                     sidx, didx, rows_a, rows_b, acc, gsem_a, gsem_b):
    c = lax.axis_index("c")
    s = lax.axis_index("s")
    nch = jnp.where(c == 0, CH_A, CH_B)
    start = c * (16 * CH_A) + s * nch

    @pl.when(c == 0)
    def _():
        pltpu.sync_copy(zrow_hbm, acc.at[pl.ds(s * RPT, RPT)])

    plsc.subcore_barrier()

    def blk(b, carry):
        base = start + b * CB
        pltpu.sync_copy(src_hbm.at[pl.ds(base, CB)], sidx)
        pltpu.sync_copy(dst_hbm.at[pl.ds(base, CB)], didx)
        pltpu.async_copy(tab_hbm.at[sidx.at[0]], rows_a, gsem_a)

        def pair(t, carry2):
            j0 = 2 * t
            j1 = j0 + 1
            pltpu.make_async_copy(tab_hbm.at[sidx.at[j0]], rows_a,
                                  gsem_a).wait()
            pltpu.async_copy(tab_hbm.at[sidx.at[j1]], rows_b, gsem_b)
            pltpu.sync_copy(rows_a, acc.at[didx.at[j0]], add=True)
            pltpu.make_async_copy(tab_hbm.at[sidx.at[j1]], rows_b,
                                  gsem_b).wait()

            @pl.when(j0 + 2 < CB)
            def _():
                pltpu.async_copy(tab_hbm.at[sidx.at[j0 + 2]], rows_a,
                                 gsem_a)

            pltpu.sync_copy(rows_b, acc.at[didx.at[j1]], add=True)
            return carry2

        return lax.fori_loop(0, CB // 2, pair, carry)

    lax.fori_loop(0, nch // CB, blk, 0)
    plsc.subcore_barrier()

    @pl.when(c == 0)
    def _():
        pltpu.sync_copy(acc.at[pl.ds(s * RPT, RPT)],
                        out_hbm.at[pl.ds(s * RPT, RPT)])


_deg_kernel = functools.partial(
    pl.kernel,
    out_type=jax.ShapeDtypeStruct((NC, NP, D), jnp.float32),
    mesh=_mesh,
    scratch_types=[
        pltpu.VMEM((CH2, C), jnp.int32),
        pltpu.VMEM((C, D), jnp.float32),
        pltpu.VMEM_SHARED((NP, D), jnp.float32),
    ],
)(_deg_kernel_body)

_agg_kernel = functools.partial(
    pl.kernel,
    out_type=jax.ShapeDtypeStruct((NP, D), jnp.float32),
    mesh=_mesh,
    scratch_types=[
        pltpu.VMEM((CB, C), jnp.int32),
        pltpu.VMEM((CB, C), jnp.int32),
        pltpu.VMEM((C, D), jnp.float32),
        pltpu.VMEM((C, D), jnp.float32),
        pltpu.VMEM_SHARED((NP, D), jnp.float32),
        pltpu.SemaphoreType.DMA,
        pltpu.SemaphoreType.DMA,
    ],
)(_agg_kernel_body)


_R = 1024


def _norm_from(deg_blk):
    return lax.rsqrt(jnp.maximum(deg_blk[:, :1], 1.0))


def _mm_scale_body(x_ref, w_ref, h_ref, o_ref):
    ns = _norm_from(h_ref[0])
    o_ref[...] = jnp.dot(x_ref[...], w_ref[...],
                         preferred_element_type=jnp.float32) * ns


def _post1_body(p_ref, h_ref, b_ref, w_ref, o_ref):
    nd = _norm_from(h_ref[1])
    ns = _norm_from(h_ref[0])
    hmid = jnp.maximum(p_ref[...] * nd + b_ref[...], 0.0)
    o_ref[...] = jnp.dot(hmid, w_ref[...],
                         preferred_element_type=jnp.float32) * ns


def _post2_body(p_ref, h_ref, b_ref, o_ref):
    nd = _norm_from(h_ref[1])
    o_ref[...] = p_ref[...] * nd + b_ref[...]


_hist_spec = pl.BlockSpec((2, _R, D), lambda i: (0, i, 0))
_part_spec = pl.BlockSpec((_R, D), lambda i: (i, 0))
_row_spec = pl.BlockSpec((_R, D), lambda i: (i, 0))
_w_spec = pl.BlockSpec((D, D), lambda i: (0, 0))
_b_spec = pl.BlockSpec((1, D), lambda i: (0, 0))
_grid = (NP // _R,)

_mm_scale = pl.pallas_call(
    _mm_scale_body,
    grid=_grid,
    in_specs=[_row_spec, _w_spec, _hist_spec],
    out_specs=_row_spec,
    out_shape=jax.ShapeDtypeStruct((NP, D), jnp.float32),
)

_post1 = pl.pallas_call(
    _post1_body,
    grid=_grid,
    in_specs=[_part_spec, _hist_spec, _b_spec, _w_spec],
    out_specs=_row_spec,
    out_shape=jax.ShapeDtypeStruct((NP, D), jnp.float32),
)

_post2 = pl.pallas_call(
    _post2_body,
    grid=_grid,
    in_specs=[_part_spec, _hist_spec, _b_spec],
    out_specs=_row_spec,
    out_shape=jax.ShapeDtypeStruct((NP, D), jnp.float32),
)


def kernel(x, edge_index, W1, b1, W2, b2):
    epad = jnp.pad(edge_index, ((0, 0), (0, EP - E)), constant_values=N)
    src2 = jnp.pad(epad[0].reshape(EP // C, C), ((0, CH_A - CH_B), (0, 0)),
                   constant_values=N)
    dst2 = jnp.pad(epad[1].reshape(EP // C, C), ((0, CH_A - CH_B), (0, 0)),
                   constant_values=N)
    xp = jnp.pad(x, ((0, NP - N), (0, 0)))
    zrow = jnp.zeros((RPT, D), jnp.float32)
    b1r = b1.reshape(1, D)
    b2r = b2.reshape(1, D)

    e2 = epad.reshape(2, NS, CH2, C)
    ones_rows = jnp.ones((C, D), jnp.float32)
    hist = _deg_kernel(e2, ones_rows, zrow)
    g1 = _mm_scale(xp, W1, hist)
    p1 = _agg_kernel(src2, dst2, g1, zrow)
    g2 = _post1(p1, hist, b1r, W2)
    p2 = _agg_kernel(src2, dst2, g2, zrow)
    return _post2(p2, hist, b2r)[:N]


# speedup vs baseline: 1.4473x; 1.4473x over previous
import functools

import jax
import jax.numpy as jnp
from jax import lax
from jax.experimental import pallas as pl
from jax.experimental.pallas import tpu as pltpu
from jax.experimental.pallas import tpu_sc as plsc

N = 10000
NP = 10240
E = 320000
EP = 327680
D = 128
NC = 2
NS = 16
NW = NC * NS
C = 128
CH = EP // (NW * C)
RPT = NP // NS

_mesh = plsc.VectorSubcoreMesh(core_axis_name="c", subcore_axis_name="s")


CH2 = EP // (NS * C)


def _deg_kernel_body(idx_hbm, ones_hbm, zrow_hbm, out_hbm,
                     cidx, ones_v, hist):
    c = lax.axis_index("c")
    s = lax.axis_index("s")
    pltpu.sync_copy(zrow_hbm, hist.at[pl.ds(s * RPT, RPT)])
    pltpu.sync_copy(ones_hbm, ones_v)
    pltpu.sync_copy(idx_hbm.at[c, s], cidx)
    plsc.subcore_barrier()

    def body(j, carry):
        pltpu.sync_copy(ones_v, hist.at[cidx.at[j]], add=True)
        return carry

    lax.fori_loop(0, CH2, body, 0)
    plsc.subcore_barrier()
    pltpu.sync_copy(hist.at[pl.ds(s * RPT, RPT)],
                    out_hbm.at[c, pl.ds(s * RPT, RPT)])


CH_A = 144
CH_B = 16
M2 = 16 * CH_A + 16 * CH_B + (CH_A - CH_B)


CB = 16


def _agg_kernel_body(src_hbm, dst_hbm, tab_hbm, zrow_hbm, out_hbm,
                     sidx, didx, rows_a, rows_b, acc, gsem_a, gsem_b):
    c = lax.axis_index("c")
    s = lax.axis_index("s")
    nch = jnp.where(c == 0, CH_A, CH_B)
    start = c * (16 * CH_A) + s * nch
    pltpu.sync_copy(zrow_hbm, acc.at[pl.ds(s * RPT, RPT)])
    plsc.subcore_barrier()

    def blk(b, carry):
        base = start + b * CB
        pltpu.sync_copy(src_hbm.at[pl.ds(base, CB)], sidx)
        pltpu.sync_copy(dst_hbm.at[pl.ds(base, CB)], didx)
        pltpu.async_copy(tab_hbm.at[sidx.at[0]], rows_a, gsem_a)

        def pair(t, carry2):
            j0 = 2 * t
            j1 = j0 + 1
            pltpu.make_async_copy(tab_hbm.at[sidx.at[j0]], rows_a,
                                  gsem_a).wait()
            pltpu.async_copy(tab_hbm.at[sidx.at[j1]], rows_b, gsem_b)
            pltpu.sync_copy(rows_a, acc.at[didx.at[j0]], add=True)
            pltpu.make_async_copy(tab_hbm.at[sidx.at[j1]], rows_b,
                                  gsem_b).wait()

            @pl.when(j0 + 2 < CB)
            def _():
                pltpu.async_copy(tab_hbm.at[sidx.at[j0 + 2]], rows_a,
                                 gsem_a)

            pltpu.sync_copy(rows_b, acc.at[didx.at[j1]], add=True)
            return carry2

        return lax.fori_loop(0, CB // 2, pair, carry)

    lax.fori_loop(0, nch // CB, blk, 0)
    plsc.subcore_barrier()
    pltpu.sync_copy(acc.at[pl.ds(s * RPT, RPT)],
                    out_hbm.at[c, pl.ds(s * RPT, RPT)])


_deg_kernel = functools.partial(
    pl.kernel,
    out_type=jax.ShapeDtypeStruct((NC, NP, D), jnp.float32),
    mesh=_mesh,
    scratch_types=[
        pltpu.VMEM((CH2, C), jnp.int32),
        pltpu.VMEM((C, D), jnp.float32),
        pltpu.VMEM_SHARED((NP, D), jnp.float32),
    ],
)(_deg_kernel_body)

_agg_kernel = functools.partial(
    pl.kernel,
    out_type=jax.ShapeDtypeStruct((NC, NP, D), jnp.float32),
    mesh=_mesh,
    scratch_types=[
        pltpu.VMEM((CB, C), jnp.int32),
        pltpu.VMEM((CB, C), jnp.int32),
        pltpu.VMEM((C, D), jnp.float32),
        pltpu.VMEM((C, D), jnp.float32),
        pltpu.VMEM_SHARED((NP, D), jnp.float32),
        pltpu.SemaphoreType.DMA,
        pltpu.SemaphoreType.DMA,
    ],
)(_agg_kernel_body)


_R = 1024


def _norm_from(deg_blk):
    return lax.rsqrt(jnp.maximum(deg_blk[:, :1], 1.0))


def _mm_scale_body(x_ref, w_ref, h_ref, o_ref):
    ns = _norm_from(h_ref[0])
    o_ref[...] = jnp.dot(x_ref[...], w_ref[...],
                         preferred_element_type=jnp.float32) * ns


def _post1_body(p_ref, h_ref, b_ref, w_ref, o_ref):
    nd = _norm_from(h_ref[1])
    ns = _norm_from(h_ref[0])
    hmid = jnp.maximum((p_ref[0] + p_ref[1]) * nd + b_ref[...], 0.0)
    o_ref[...] = jnp.dot(hmid, w_ref[...],
                         preferred_element_type=jnp.float32) * ns


def _post2_body(p_ref, h_ref, b_ref, o_ref):
    nd = _norm_from(h_ref[1])
    o_ref[...] = (p_ref[0] + p_ref[1]) * nd + b_ref[...]


_hist_spec = pl.BlockSpec((2, _R, D), lambda i: (0, i, 0))
_part_spec = pl.BlockSpec((NC, _R, D), lambda i: (0, i, 0))
_row_spec = pl.BlockSpec((_R, D), lambda i: (i, 0))
_w_spec = pl.BlockSpec((D, D), lambda i: (0, 0))
_b_spec = pl.BlockSpec((1, D), lambda i: (0, 0))
_grid = (NP // _R,)

_mm_scale = pl.pallas_call(
    _mm_scale_body,
    grid=_grid,
    in_specs=[_row_spec, _w_spec, _hist_spec],
    out_specs=_row_spec,
    out_shape=jax.ShapeDtypeStruct((NP, D), jnp.float32),
)

_post1 = pl.pallas_call(
    _post1_body,
    grid=_grid,
    in_specs=[_part_spec, _hist_spec, _b_spec, _w_spec],
    out_specs=_row_spec,
    out_shape=jax.ShapeDtypeStruct((NP, D), jnp.float32),
)

_post2 = pl.pallas_call(
    _post2_body,
    grid=_grid,
    in_specs=[_part_spec, _hist_spec, _b_spec],
    out_specs=_row_spec,
    out_shape=jax.ShapeDtypeStruct((NP, D), jnp.float32),
)


def kernel(x, edge_index, W1, b1, W2, b2):
    epad = jnp.pad(edge_index, ((0, 0), (0, EP - E)), constant_values=N)
    src2 = jnp.pad(epad[0].reshape(EP // C, C), ((0, CH_A - CH_B), (0, 0)),
                   constant_values=N)
    dst2 = jnp.pad(epad[1].reshape(EP // C, C), ((0, CH_A - CH_B), (0, 0)),
                   constant_values=N)
    xp = jnp.pad(x, ((0, NP - N), (0, 0)))
    zrow = jnp.zeros((RPT, D), jnp.float32)
    b1r = b1.reshape(1, D)
    b2r = b2.reshape(1, D)

    e2 = epad.reshape(2, NS, CH2, C)
    ones_rows = jnp.ones((C, D), jnp.float32)
    hist = _deg_kernel(e2, ones_rows, zrow)
    g1 = _mm_scale(xp, W1, hist)
    p1 = _agg_kernel(src2, dst2, g1, zrow)
    g2 = _post1(p1, hist, b1r, W2)
    p2 = _agg_kernel(src2, dst2, g2, zrow)
    return _post2(p2, hist, b2r)[:N]
